# baseline (speedup 1.0000x reference)
import functools

import jax
import jax.numpy as jnp
from jax import lax
from jax.experimental import pallas as pl
from jax.experimental.pallas import tpu as pltpu
from jax.experimental.pallas import tpu_sc as plsc

_VOCAB = 100000
_EMB = 128
_HID = 768
_MAXPOS = 4096
_TYPES = 4
_B, _S = 4, 4096
_TOK = _B * _S
_EPS = 1e-12

_SPLITS = ((0, 1), (1, 3))


_NC, _NS = 2, 16
_NW = _NC * _NS
_CHUNK = 128
_IDXROWS_PER_BATCH = _S // _CHUNK


def _sc_gather_body(b0, nb, table_hbm, idx_hbm, out_hbm, idx_v, rows_v,
                    gsem, wsem):
    del b0
    rows_per_w = nb * _S // _NW
    nchunk = rows_per_w // _CHUNK
    wid = lax.axis_index("s") * _NC + lax.axis_index("c")
    pltpu.sync_copy(idx_hbm.at[wid], idx_v)
    gathers = []
    for j in range(nchunk):
        gathers.append(
            pltpu.async_copy(
                table_hbm.at[idx_v.at[j]],
                rows_v.at[pl.ds(j * _CHUNK, _CHUNK)],
                gsem,
            )
        )
    writes = []
    for j in range(nchunk):
        gathers[j].wait()
        writes.append(
            pltpu.async_copy(
                rows_v.at[pl.ds(j * _CHUNK, _CHUNK)],
                out_hbm.at[pl.ds(wid * rows_per_w + j * _CHUNK, _CHUNK)],
                wsem,
            )
        )
    for cp in writes:
        cp.wait()


@functools.cache
def _sc_gather(b0, nb):
    rows_per_w = nb * _S // _NW
    return functools.partial(
        pl.kernel,
        mesh=plsc.VectorSubcoreMesh(core_axis_name="c", subcore_axis_name="s"),
        out_type=jax.ShapeDtypeStruct((nb * _S, _EMB), jnp.float32),
        scratch_types=[
            pltpu.VMEM((rows_per_w // _CHUNK, _CHUNK), jnp.int32),
            pltpu.VMEM((rows_per_w, _EMB), jnp.float32),
            pltpu.SemaphoreType.DMA,
            pltpu.SemaphoreType.DMA,
        ],
    )(functools.partial(_sc_gather_body, b0, nb))


_BT = 2048
_JB = _MAXPOS // _BT


def _tc_body(g_ref, p_ref, t_ref, te_ref, gam_ref, bet_ref, w_ref, b_ref,
             *prev_and_out):
    o_ref = prev_and_out[-1]
    tid = t_ref[...]
    oh = (
        lax.broadcasted_iota(jnp.int32, (8, _BT), 0) == tid
    ).astype(jnp.float32)
    te = lax.dot_general(
        oh, te_ref[...], (((0,), (0,)), ((), ())),
        preferred_element_type=jnp.float32,
    )
    acc = g_ref[...] + p_ref[...] + te
    mu = jnp.mean(acc, axis=1, keepdims=True)
    d = acc - mu
    var = jnp.mean(d * d, axis=1, keepdims=True)
    y = d * lax.rsqrt(var + _EPS) * gam_ref[...] + bet_ref[...]
    o_ref[...] = (
        jnp.dot(y, w_ref[...], preferred_element_type=jnp.float32) + b_ref[...]
    )


@functools.cache
def _tc_fused(b0, nb):
    def row(j, bi):
        return (b0 + bi) * _JB + j

    specs = [
        pl.BlockSpec((_BT, _EMB), lambda j, bi: (bi * _JB + j, 0)),
        pl.BlockSpec((_BT, _EMB), lambda j, bi: (j, 0)),
        pl.BlockSpec((1, _BT), lambda j, bi: (0, row(j, bi))),
        pl.BlockSpec((8, _EMB), lambda j, bi: (0, 0)),
        pl.BlockSpec((1, _EMB), lambda j, bi: (0, 0)),
        pl.BlockSpec((1, _EMB), lambda j, bi: (0, 0)),
        pl.BlockSpec((_EMB, _HID), lambda j, bi: (0, 0)),
        pl.BlockSpec((1, _HID), lambda j, bi: (0, 0)),
    ]
    aliases = {}
    if b0 > 0:
        specs.append(pl.BlockSpec(memory_space=pl.ANY))
        aliases = {8: 0}
    return pl.pallas_call(
        _tc_body,
        grid=(_JB, nb),
        in_specs=specs,
        out_specs=pl.BlockSpec((_BT, _HID), lambda j, bi: (row(j, bi), 0)),
        out_shape=jax.ShapeDtypeStruct((_TOK, _HID), jnp.float32),
        input_output_aliases=aliases,
    )


def kernel(input_ids, type_ids, word_emb, pos_emb, type_emb, gamma, beta, W, b):
    ids32 = input_ids.astype(jnp.int32)
    te_pad = jnp.zeros((8, _EMB), jnp.float32).at[:_TYPES].set(type_emb)
    tid_row = type_ids.astype(jnp.int32).reshape(1, _TOK)
    gamma2 = gamma.reshape(1, _EMB)
    beta2 = beta.reshape(1, _EMB)
    b2 = b.reshape(1, _HID)

    gath = []
    for b0, nb in _SPLITS:
        ids3d = ids32[b0:b0 + nb].reshape(_NW, nb * _S // _NW // _CHUNK, _CHUNK)
        gath.append(_sc_gather(b0, nb)(word_emb, ids3d))
    out = None
    for (b0, nb), g in zip(_SPLITS, gath):
        args = [g, pos_emb, tid_row, te_pad, gamma2, beta2, W, b2]
        if b0 > 0:
            args.append(out)
        out = _tc_fused(b0, nb)(*args)
    return out.reshape(_B, _S, _HID)


# speedup vs baseline: 1.0592x; 1.0592x over previous
import functools

import jax
import jax.numpy as jnp
from jax import lax
from jax.experimental import pallas as pl
from jax.experimental.pallas import tpu as pltpu
from jax.experimental.pallas import tpu_sc as plsc

_VOCAB = 100000
_EMB = 128
_HID = 768
_MAXPOS = 4096
_TYPES = 4
_B, _S = 4, 4096
_TOK = _B * _S
_EPS = 1e-12

_SPLITS = ((0, 4),)


_NC, _NS = 2, 16
_NW = _NC * _NS
_CHUNK = 64
_IDXROWS_PER_BATCH = _S // _CHUNK


def _sc_gather_body(b0, nb, table_hbm, idx_hbm, out_hbm, idx_v, rows_v,
                    gsem, wsem):
    del b0
    rows_per_w = nb * _S // _NW
    nchunk = rows_per_w // _CHUNK
    wid = lax.axis_index("s") * _NC + lax.axis_index("c")
    pltpu.sync_copy(idx_hbm.at[wid], idx_v)
    gathers = []
    for j in range(nchunk):
        gathers.append(
            pltpu.async_copy(
                table_hbm.at[idx_v.at[j]],
                rows_v.at[pl.ds(j * _CHUNK, _CHUNK)],
                gsem,
            )
        )
    writes = []
    for j in range(nchunk):
        gathers[j].wait()
        writes.append(
            pltpu.async_copy(
                rows_v.at[pl.ds(j * _CHUNK, _CHUNK)],
                out_hbm.at[pl.ds(wid * rows_per_w + j * _CHUNK, _CHUNK)],
                wsem,
            )
        )
    for cp in writes:
        cp.wait()


@functools.cache
def _sc_gather(b0, nb):
    rows_per_w = nb * _S // _NW
    return functools.partial(
        pl.kernel,
        mesh=plsc.VectorSubcoreMesh(core_axis_name="c", subcore_axis_name="s"),
        out_type=jax.ShapeDtypeStruct((nb * _S, _EMB), jnp.float32),
        scratch_types=[
            pltpu.VMEM((rows_per_w // _CHUNK, _CHUNK), jnp.int32),
            pltpu.VMEM((rows_per_w, _EMB), jnp.float32),
            pltpu.SemaphoreType.DMA,
            pltpu.SemaphoreType.DMA,
        ],
    )(functools.partial(_sc_gather_body, b0, nb))


_BT = 2048
_JB = _MAXPOS // _BT


def _tc_body(g_ref, p_ref, t_ref, te_ref, gam_ref, bet_ref, w_ref, b_ref,
             *prev_and_out):
    o_ref = prev_and_out[-1]
    tid = t_ref[...]
    oh = (
        lax.broadcasted_iota(jnp.int32, (8, _BT), 0) == tid
    ).astype(jnp.float32)
    te = lax.dot_general(
        oh, te_ref[...], (((0,), (0,)), ((), ())),
        preferred_element_type=jnp.float32,
    )
    acc = g_ref[...] + p_ref[...] + te
    mu = jnp.mean(acc, axis=1, keepdims=True)
    d = acc - mu
    var = jnp.mean(d * d, axis=1, keepdims=True)
    y = d * lax.rsqrt(var + _EPS) * gam_ref[...] + bet_ref[...]
    o_ref[...] = (
        jnp.dot(y, w_ref[...], preferred_element_type=jnp.float32) + b_ref[...]
    )


@functools.cache
def _tc_fused(b0, nb):
    def row(j, bi):
        return (b0 + bi) * _JB + j

    specs = [
        pl.BlockSpec((_BT, _EMB), lambda j, bi: (bi * _JB + j, 0)),
        pl.BlockSpec((_BT, _EMB), lambda j, bi: (j, 0)),
        pl.BlockSpec((1, _BT), lambda j, bi: (0, row(j, bi))),
        pl.BlockSpec((8, _EMB), lambda j, bi: (0, 0)),
        pl.BlockSpec((1, _EMB), lambda j, bi: (0, 0)),
        pl.BlockSpec((1, _EMB), lambda j, bi: (0, 0)),
        pl.BlockSpec((_EMB, _HID), lambda j, bi: (0, 0)),
        pl.BlockSpec((1, _HID), lambda j, bi: (0, 0)),
    ]
    aliases = {}
    if b0 > 0:
        specs.append(pl.BlockSpec(memory_space=pl.ANY))
        aliases = {8: 0}
    return pl.pallas_call(
        _tc_body,
        grid=(_JB, nb),
        in_specs=specs,
        out_specs=pl.BlockSpec((_BT, _HID), lambda j, bi: (row(j, bi), 0)),
        out_shape=jax.ShapeDtypeStruct((_TOK, _HID), jnp.float32),
        input_output_aliases=aliases,
    )


def kernel(input_ids, type_ids, word_emb, pos_emb, type_emb, gamma, beta, W, b):
    ids32 = input_ids.astype(jnp.int32)
    te_pad = jnp.zeros((8, _EMB), jnp.float32).at[:_TYPES].set(type_emb)
    tid_row = type_ids.astype(jnp.int32).reshape(1, _TOK)
    gamma2 = gamma.reshape(1, _EMB)
    beta2 = beta.reshape(1, _EMB)
    b2 = b.reshape(1, _HID)

    gath = []
    for b0, nb in _SPLITS:
        ids3d = ids32[b0:b0 + nb].reshape(_NW, nb * _S // _NW // _CHUNK, _CHUNK)
        gath.append(_sc_gather(b0, nb)(word_emb, ids3d))
    out = None
    for (b0, nb), g in zip(_SPLITS, gath):
        args = [g, pos_emb, tid_row, te_pad, gamma2, beta2, W, b2]
        if b0 > 0:
            args.append(out)
        out = _tc_fused(b0, nb)(*args)
    return out.reshape(_B, _S, _HID)


# speedup vs baseline: 1.0637x; 1.0042x over previous
import functools

import jax
import jax.numpy as jnp
from jax import lax
from jax.experimental import pallas as pl
from jax.experimental.pallas import tpu as pltpu
from jax.experimental.pallas import tpu_sc as plsc

_VOCAB = 100000
_EMB = 128
_HID = 768
_MAXPOS = 4096
_TYPES = 4
_B, _S = 4, 4096
_TOK = _B * _S
_EPS = 1e-12

_SPLITS = ((0, 4),)


_NC, _NS = 2, 16
_NW = _NC * _NS
_CHUNK = 128
_IDXROWS_PER_BATCH = _S // _CHUNK


def _sc_gather_body(b0, nb, table_hbm, idx_hbm, out_hbm, idx_v, rows_v,
                    gsem, wsem):
    del b0
    rows_per_w = nb * _S // _NW
    nchunk = rows_per_w // _CHUNK
    wid = lax.axis_index("s") * _NC + lax.axis_index("c")
    pltpu.sync_copy(idx_hbm.at[wid], idx_v)
    gathers = []
    for j in range(nchunk):
        gathers.append(
            pltpu.async_copy(
                table_hbm.at[idx_v.at[j]],
                rows_v.at[pl.ds(j * _CHUNK, _CHUNK)],
                gsem,
            )
        )
    writes = []
    for j in range(nchunk):
        gathers[j].wait()
        writes.append(
            pltpu.async_copy(
                rows_v.at[pl.ds(j * _CHUNK, _CHUNK)],
                out_hbm.at[pl.ds(wid * rows_per_w + j * _CHUNK, _CHUNK)],
                wsem,
            )
        )
    for cp in writes:
        cp.wait()


@functools.cache
def _sc_gather(b0, nb):
    rows_per_w = nb * _S // _NW
    return functools.partial(
        pl.kernel,
        mesh=plsc.VectorSubcoreMesh(core_axis_name="c", subcore_axis_name="s"),
        out_type=jax.ShapeDtypeStruct((nb * _S, _EMB), jnp.float32),
        scratch_types=[
            pltpu.VMEM((rows_per_w // _CHUNK, _CHUNK), jnp.int32),
            pltpu.VMEM((rows_per_w, _EMB), jnp.float32),
            pltpu.SemaphoreType.DMA,
            pltpu.SemaphoreType.DMA,
        ],
    )(functools.partial(_sc_gather_body, b0, nb))


_BT = 2048
_JB = _MAXPOS // _BT


def _tc_body(g_ref, p_ref, t_ref, te_ref, gam_ref, bet_ref, w_ref, b_ref,
             *prev_and_out):
    o_ref = prev_and_out[-1]
    tid = t_ref[...]
    oh = (
        lax.broadcasted_iota(jnp.int32, (8, _BT), 0) == tid
    ).astype(jnp.float32)
    te = lax.dot_general(
        oh, te_ref[...], (((0,), (0,)), ((), ())),
        preferred_element_type=jnp.float32,
    )
    acc = g_ref[...] + p_ref[...] + te
    mu = jnp.mean(acc, axis=1, keepdims=True)
    d = acc - mu
    var = jnp.mean(d * d, axis=1, keepdims=True)
    y = d * lax.rsqrt(var + _EPS) * gam_ref[...] + bet_ref[...]
    o_ref[...] = (
        jnp.dot(y, w_ref[...], preferred_element_type=jnp.float32) + b_ref[...]
    )


@functools.cache
def _tc_fused(b0, nb):
    def row(j, bi):
        return (b0 + bi) * _JB + j

    specs = [
        pl.BlockSpec((_BT, _EMB), lambda j, bi: (bi * _JB + j, 0)),
        pl.BlockSpec((_BT, _EMB), lambda j, bi: (j, 0)),
        pl.BlockSpec((1, _BT), lambda j, bi: (0, row(j, bi))),
        pl.BlockSpec((8, _EMB), lambda j, bi: (0, 0)),
        pl.BlockSpec((1, _EMB), lambda j, bi: (0, 0)),
        pl.BlockSpec((1, _EMB), lambda j, bi: (0, 0)),
        pl.BlockSpec((_EMB, _HID), lambda j, bi: (0, 0)),
        pl.BlockSpec((1, _HID), lambda j, bi: (0, 0)),
    ]
    aliases = {}
    if b0 > 0:
        specs.append(pl.BlockSpec(memory_space=pl.ANY))
        aliases = {8: 0}
    return pl.pallas_call(
        _tc_body,
        grid=(_JB, nb),
        in_specs=specs,
        out_specs=pl.BlockSpec((_BT, _HID), lambda j, bi: (row(j, bi), 0)),
        out_shape=jax.ShapeDtypeStruct((_TOK, _HID), jnp.float32),
        input_output_aliases=aliases,
    )


def kernel(input_ids, type_ids, word_emb, pos_emb, type_emb, gamma, beta, W, b):
    ids32 = input_ids.astype(jnp.int32)
    te_pad = jnp.zeros((8, _EMB), jnp.float32).at[:_TYPES].set(type_emb)
    tid_row = type_ids.astype(jnp.int32).reshape(1, _TOK)
    gamma2 = gamma.reshape(1, _EMB)
    beta2 = beta.reshape(1, _EMB)
    b2 = b.reshape(1, _HID)

    gath = []
    for b0, nb in _SPLITS:
        ids3d = ids32[b0:b0 + nb].reshape(_NW, nb * _S // _NW // _CHUNK, _CHUNK)
        gath.append(_sc_gather(b0, nb)(word_emb, ids3d))
    out = None
    for (b0, nb), g in zip(_SPLITS, gath):
        args = [g, pos_emb, tid_row, te_pad, gamma2, beta2, W, b2]
        if b0 > 0:
            args.append(out)
        out = _tc_fused(b0, nb)(*args)
    return out.reshape(_B, _S, _HID)


# speedup vs baseline: 1.0651x; 1.0013x over previous
import functools

import jax
import jax.numpy as jnp
from jax import lax
from jax.experimental import pallas as pl
from jax.experimental.pallas import tpu as pltpu
from jax.experimental.pallas import tpu_sc as plsc

_EMB = 128
_HID = 768
_MAXPOS = 4096
_TYPES = 4
_B, _S = 4, 4096
_TOK = _B * _S
_EPS = 1e-12


_NC, _NS = 2, 16
_NW = _NC * _NS
_ROWS_PER_W = _TOK // _NW
_CHUNK = 128
_NCHUNK = _ROWS_PER_W // _CHUNK


def _sc_gather_body(table_hbm, idx_hbm, out_hbm, idx_v, rows_v, gsem, wsem):
    wid = lax.axis_index("s") * _NC + lax.axis_index("c")
    pltpu.sync_copy(idx_hbm.at[wid], idx_v)
    gathers = []
    for j in range(_NCHUNK):
        gathers.append(
            pltpu.async_copy(
                table_hbm.at[idx_v.at[j]],
                rows_v.at[pl.ds(j * _CHUNK, _CHUNK)],
                gsem,
            )
        )
    writes = []
    for j in range(_NCHUNK):
        gathers[j].wait()
        writes.append(
            pltpu.async_copy(
                rows_v.at[pl.ds(j * _CHUNK, _CHUNK)],
                out_hbm.at[pl.ds(wid * _ROWS_PER_W + j * _CHUNK, _CHUNK)],
                wsem,
            )
        )
    for cp in writes:
        cp.wait()


@functools.cache
def _sc_gather():
    return functools.partial(
        pl.kernel,
        mesh=plsc.VectorSubcoreMesh(core_axis_name="c", subcore_axis_name="s"),
        out_type=jax.ShapeDtypeStruct((_TOK, _EMB), jnp.float32),
        scratch_types=[
            pltpu.VMEM((_NCHUNK, _CHUNK), jnp.int32),
            pltpu.VMEM((_ROWS_PER_W, _EMB), jnp.float32),
            pltpu.SemaphoreType.DMA,
            pltpu.SemaphoreType.DMA,
        ],
    )(_sc_gather_body)


_BT = 2048
_JB = _MAXPOS // _BT


def _tc_body(g_ref, p_ref, t_ref, te_ref, gam_ref, bet_ref, w_ref, b_ref,
             o_ref):
    tid = t_ref[...]
    oh = (
        lax.broadcasted_iota(jnp.int32, (8, _BT), 0) == tid
    ).astype(jnp.float32)
    te = lax.dot_general(
        oh, te_ref[...], (((0,), (0,)), ((), ())),
        preferred_element_type=jnp.float32,
    )
    acc = g_ref[...] + p_ref[...] + te
    mu = jnp.mean(acc, axis=1, keepdims=True)
    d = acc - mu
    var = jnp.mean(d * d, axis=1, keepdims=True)
    y = d * lax.rsqrt(var + _EPS) * gam_ref[...] + bet_ref[...]
    o_ref[...] = (
        jnp.dot(y, w_ref[...], preferred_element_type=jnp.float32) + b_ref[...]
    )


@functools.cache
def _tc_fused():
    return pl.pallas_call(
        _tc_body,
        grid=(_JB, _B),
        in_specs=[
            pl.BlockSpec((_BT, _EMB), lambda j, bi: (bi * _JB + j, 0)),
            pl.BlockSpec((_BT, _EMB), lambda j, bi: (j, 0)),
            pl.BlockSpec((1, _BT), lambda j, bi: (0, bi * _JB + j)),
            pl.BlockSpec((8, _EMB), lambda j, bi: (0, 0)),
            pl.BlockSpec((1, _EMB), lambda j, bi: (0, 0)),
            pl.BlockSpec((1, _EMB), lambda j, bi: (0, 0)),
            pl.BlockSpec((_EMB, _HID), lambda j, bi: (0, 0)),
            pl.BlockSpec((1, _HID), lambda j, bi: (0, 0)),
        ],
        out_specs=pl.BlockSpec((_BT, _HID), lambda j, bi: (bi * _JB + j, 0)),
        out_shape=jax.ShapeDtypeStruct((_TOK, _HID), jnp.float32),
    )


def kernel(input_ids, type_ids, word_emb, pos_emb, type_emb, gamma, beta, W, b):
    ids3d = input_ids.astype(jnp.int32).reshape(_NW, _NCHUNK, _CHUNK)
    gathered = _sc_gather()(word_emb, ids3d)
    te_pad = jnp.zeros((8, _EMB), jnp.float32).at[:_TYPES].set(type_emb)
    out = _tc_fused()(
        gathered,
        pos_emb,
        type_ids.astype(jnp.int32).reshape(1, _TOK),
        te_pad,
        gamma.reshape(1, _EMB),
        beta.reshape(1, _EMB),
        W,
        b.reshape(1, _HID),
    )
    return out.reshape(_B, _S, _HID)


# speedup vs baseline: 1.0696x; 1.0042x over previous
import functools

import jax
import jax.numpy as jnp
from jax import lax
from jax.experimental import pallas as pl
from jax.experimental.pallas import tpu as pltpu
from jax.experimental.pallas import tpu_sc as plsc

_EMB = 128
_HID = 768
_MAXPOS = 4096
_TYPES = 4
_B, _S = 4, 4096
_TOK = _B * _S
_EPS = 1e-12


_NC, _NS = 2, 16
_NW = _NC * _NS
_ROWS_PER_W = _TOK // _NW
_CHUNK = 128
_NCHUNK = _ROWS_PER_W // _CHUNK


def _sc_gather_body(table_hbm, idx_hbm, out_hbm, idx_v, rows_v, gsem, wsem):
    wid = lax.axis_index("s") * _NC + lax.axis_index("c")
    pltpu.sync_copy(idx_hbm.at[:, pl.ds(wid * _CHUNK, _CHUNK)], idx_v)
    gathers = []
    for j in range(_NCHUNK):
        gathers.append(
            pltpu.async_copy(
                table_hbm.at[idx_v.at[j]],
                rows_v.at[pl.ds(j * _CHUNK, _CHUNK)],
                gsem,
            )
        )
    writes = []
    for j in range(_NCHUNK):
        gathers[j].wait()
        writes.append(
            pltpu.async_copy(
                rows_v.at[pl.ds(j * _CHUNK, _CHUNK)],
                out_hbm.at[pl.ds(j * _S + wid * _CHUNK, _CHUNK)],
                wsem,
            )
        )
    for cp in writes:
        cp.wait()


@functools.cache
def _sc_gather():
    return functools.partial(
        pl.kernel,
        mesh=plsc.VectorSubcoreMesh(core_axis_name="c", subcore_axis_name="s"),
        out_type=jax.ShapeDtypeStruct((_TOK, _EMB), jnp.float32),
        scratch_types=[
            pltpu.VMEM((_NCHUNK, _CHUNK), jnp.int32),
            pltpu.VMEM((_ROWS_PER_W, _EMB), jnp.float32),
            pltpu.SemaphoreType.DMA,
            pltpu.SemaphoreType.DMA,
        ],
        compiler_params=pltpu.CompilerParams(use_tc_tiling_on_sc=True),
    )(_sc_gather_body)


_BT = 2048
_JB = _MAXPOS // _BT


def _tc_body(g_ref, p_ref, t_ref, te_ref, gam_ref, bet_ref, w_ref, b_ref,
             o_ref):
    tid = t_ref[...]
    oh = (
        lax.broadcasted_iota(jnp.int32, (8, _BT), 0) == tid
    ).astype(jnp.float32)
    te = lax.dot_general(
        oh, te_ref[...], (((0,), (0,)), ((), ())),
        preferred_element_type=jnp.float32,
    )
    acc = g_ref[...] + p_ref[...] + te
    mu = jnp.mean(acc, axis=1, keepdims=True)
    d = acc - mu
    var = jnp.mean(d * d, axis=1, keepdims=True)
    y = d * lax.rsqrt(var + _EPS) * gam_ref[...] + bet_ref[...]
    o_ref[...] = (
        jnp.dot(y, w_ref[...], preferred_element_type=jnp.float32) + b_ref[...]
    )


@functools.cache
def _tc_fused():
    return pl.pallas_call(
        _tc_body,
        grid=(_JB, _B),
        in_specs=[
            pl.BlockSpec((_BT, _EMB), lambda j, bi: (bi * _JB + j, 0)),
            pl.BlockSpec((_BT, _EMB), lambda j, bi: (j, 0)),
            pl.BlockSpec((1, _BT), lambda j, bi: (0, bi * _JB + j)),
            pl.BlockSpec((8, _EMB), lambda j, bi: (0, 0)),
            pl.BlockSpec((1, _EMB), lambda j, bi: (0, 0)),
            pl.BlockSpec((1, _EMB), lambda j, bi: (0, 0)),
            pl.BlockSpec((_EMB, _HID), lambda j, bi: (0, 0)),
            pl.BlockSpec((1, _HID), lambda j, bi: (0, 0)),
        ],
        out_specs=pl.BlockSpec((_BT, _HID), lambda j, bi: (bi * _JB + j, 0)),
        out_shape=jax.ShapeDtypeStruct((_TOK, _HID), jnp.float32),
    )


def kernel(input_ids, type_ids, word_emb, pos_emb, type_emb, gamma, beta, W, b):
    gathered = _sc_gather()(word_emb, input_ids.astype(jnp.int32))
    te_pad = jnp.zeros((8, _EMB), jnp.float32).at[:_TYPES].set(type_emb)
    out = _tc_fused()(
        gathered,
        pos_emb,
        type_ids.astype(jnp.int32).reshape(1, _TOK),
        te_pad,
        gamma.reshape(1, _EMB),
        beta.reshape(1, _EMB),
        W,
        b.reshape(1, _HID),
    )
    return out.reshape(_B, _S, _HID)
